# baseline (speedup 1.0000x reference)
import functools

import jax
import jax.numpy as jnp
from jax import lax
from jax.experimental import pallas as pl
from jax.experimental.pallas import tpu as pltpu
from jax.experimental.pallas import tpu_sc as plsc

_N_TABLES = 26
_BATCH = 16384
_DIM = 16
_B = _N_TABLES * _BATCH
_V = _N_TABLES * 100000

_NC = 2
_NS = 16
_NW = _NC * _NS

_FULL_TILES = _V // 128
_WP_ROWS = _V * _DIM // 128
_TILES_LO = _FULL_TILES // _NW
_TILES_EXTRA = _FULL_TILES - _TILES_LO * _NW

_G = 128
_ROWS = _B // _G
_ROWS_PER_W = _ROWS // _NW
_CBLK = _BATCH // _G


_PBUF = 3
_CB = 32
_NCH_TOT = (_WP_ROWS + _CB - 1) // _CB


@functools.partial(
    pl.kernel,
    mesh=plsc.VectorSubcoreMesh(core_axis_name="c", subcore_axis_name="s"),
    out_type=jax.ShapeDtypeStruct((_WP_ROWS, 128), jnp.float32),
    scratch_types=[
        pltpu.VMEM((_PBUF, _CB * 8, 16), jnp.float32),
        pltpu.VMEM((_PBUF, _CB, 128), jnp.float32),
        pltpu.SemaphoreType.DMA((_PBUF,)),
        pltpu.SemaphoreType.DMA((_PBUF,)),
    ],
    compiler_params=pltpu.CompilerParams(use_tc_tiling_on_sc=True, needs_layout_passes=False),
)
def _pack(w_hbm, wp_hbm, inb, outb, isems, osems):
    wid = lax.axis_index("s") * _NC + lax.axis_index("c")

    def row0_of(c):
        return jnp.minimum(c * _CB, _WP_ROWS - _CB)

    def fire_in(c, buf):
        r0 = row0_of(c)
        pltpu.async_copy(
            w_hbm.at[pl.ds(pl.multiple_of(r0 * 8, 8), _CB * 8), :],
            inb.at[buf], isems.at[buf])

    def wait_in(buf):
        pltpu.make_async_copy(
            w_hbm.at[pl.ds(0, _CB * 8), :], inb.at[buf], isems.at[buf]).wait()

    def fire_out(c, buf):
        r0 = row0_of(c)
        pltpu.async_copy(
            outb.at[buf],
            wp_hbm.at[pl.ds(pl.multiple_of(r0, 8), _CB)],
            osems.at[buf])

    def wait_out(buf):
        pltpu.make_async_copy(
            outb.at[buf], wp_hbm.at[pl.ds(0, _CB)],
            osems.at[buf]).wait()

    def compact(buf):
        for r in range(_CB * 8):
            outb[buf, r // 8, pl.ds((r % 8) * 16, 16)] = inb[buf, r, :]

    nch = (_NCH_TOT - wid + _NW - 1) // _NW

    for b in range(_PBUF - 1):
        @pl.when(b < nch)
        def _():
            fire_in(wid + b * _NW, b)

    def body(i, _):
        cur = i % _PBUF

        @pl.when(i + _PBUF - 1 < nch)
        def _():
            fire_in(wid + (i + _PBUF - 1) * _NW, (i + _PBUF - 1) % _PBUF)

        wait_in(cur)

        @pl.when(i >= _PBUF)
        def _():
            wait_out(cur)

        compact(cur)
        fire_out(wid + i * _NW, cur)
        return ()

    lax.fori_loop(0, nch, body, ())
    for b in range(_PBUF):
        @pl.when(b < nch)
        def _():
            wait_out(b)


@functools.partial(
    pl.kernel,
    mesh=plsc.VectorSubcoreMesh(core_axis_name="c", subcore_axis_name="s"),
    out_type=jax.ShapeDtypeStruct((_N_TABLES, _DIM, _BATCH), jnp.float32),
    scratch_types=[
        pltpu.VMEM((2, 128), jnp.int32),
        pltpu.VMEM((2, 128), jnp.int32),
        pltpu.VMEM((2, 128, 128), jnp.float32),
        pltpu.VMEM((2, 16, 128), jnp.float32),
        pltpu.SemaphoreType.DMA((2,)),
    ],
    compiler_params=pltpu.CompilerParams(use_tc_tiling_on_sc=True, needs_layout_passes=False),
)
def _gather(wp_hbm, idx_hbm, out_hbm, idxraw, idxrow, rowsv, outv, gsems):
    wid = lax.axis_index("s") * _NC + lax.axis_index("c")
    base = wid * _ROWS_PER_W

    def fire(blk, buf):
        pltpu.sync_copy(idx_hbm.at[blk], idxraw.at[buf])
        for kk in range(8):
            sl = pl.ds(kk * 16, 16)
            idxrow[buf, sl] = lax.shift_right_logical(idxraw[buf, sl], 3)
        pltpu.async_copy(wp_hbm.at[idxrow.at[buf]], rowsv.at[buf],
                         gsems.at[buf])

    def wait(buf):
        pltpu.make_async_copy(
            wp_hbm.at[idxrow.at[buf]], rowsv.at[buf], gsems.at[buf]).wait()

    def drain(blk, buf):
        t = blk // _CBLK
        c = blk % _CBLK
        wait(buf)

        rows2d = rowsv.at[buf]
        for j0 in range(0, 128, 16):
            vv = idxraw[buf, pl.ds(j0, 16)]
            colbase = (vv & 7) * 16
            rowix = lax.iota(jnp.int32, 16) + j0
            for d in range(_DIM):
                vec = plsc.load_gather(rows2d, [rowix, colbase + d])
                outv[buf, d, pl.ds(j0, 16)] = vec
        pltpu.sync_copy(
            outv.at[buf],
            out_hbm.at[t, :, pl.ds(pl.multiple_of(c * 128, 128), 128)])

    fire(base, 0)

    def body(i, _):
        cur = i % 2
        nxt = (i + 1) % 2

        @pl.when(i + 1 < _ROWS_PER_W)
        def _():
            fire(base + i + 1, nxt)

        drain(base + i, cur)
        return ()

    lax.fori_loop(0, _ROWS_PER_W, body, ())


def kernel(indices, offsets, weights):
    del offsets
    wp = _pack(weights)
    idx2d = indices.astype(jnp.int32).reshape(_ROWS, _G)
    out = _gather(wp, idx2d)
    return out.transpose(0, 2, 1)


# speedup vs baseline: 1.1010x; 1.1010x over previous
import functools

import jax
import jax.numpy as jnp
from jax import lax
from jax.experimental import pallas as pl
from jax.experimental.pallas import tpu as pltpu
from jax.experimental.pallas import tpu_sc as plsc

_N_TABLES = 26
_BATCH = 16384
_DIM = 16
_B = _N_TABLES * _BATCH
_V = _N_TABLES * 100000

_NC = 2
_NS = 16
_NW = _NC * _NS

_FULL_TILES = _V // 128
_WP_ROWS = _V * _DIM // 128
_TILES_LO = _FULL_TILES // _NW
_TILES_EXTRA = _FULL_TILES - _TILES_LO * _NW

_G = 128
_ROWS = _B // _G
_ROWS_PER_W = _ROWS // _NW
_CBLK = _BATCH // _G


@functools.partial(
    pl.kernel,
    mesh=plsc.VectorSubcoreMesh(core_axis_name="c", subcore_axis_name="s"),
    out_type=jax.ShapeDtypeStruct((_WP_ROWS, 128), jnp.float32),
    scratch_types=[
        pltpu.VMEM((2, 16, 128), jnp.float32),
        pltpu.VMEM((2, 16, 128), jnp.float32),
        pltpu.VMEM((64, 16), jnp.float32),
        pltpu.SemaphoreType.DMA((2,)),
    ],
    compiler_params=pltpu.CompilerParams(use_tc_tiling_on_sc=True, needs_layout_passes=False),
)
def _pack(wt_hbm, tail_hbm, wp_hbm, inb, outb, tailb, isems):
    wid = lax.axis_index("s") * _NC + lax.axis_index("c")
    ntiles = jnp.where(wid < _TILES_EXTRA, _TILES_LO + 1, _TILES_LO)
    tile0 = wid * _TILES_LO + jnp.minimum(wid, _TILES_EXTRA)

    def fire(t, buf):
        pltpu.async_copy(
            wt_hbm.at[:, pl.ds(pl.multiple_of(t * 128, 128), 128)],
            inb.at[buf], isems.at[buf])

    def wait(buf):
        pltpu.make_async_copy(
            wt_hbm.at[:, pl.ds(0, 128)], inb.at[buf], isems.at[buf]).wait()

    def transpose_tile(buf):
        def col(j, _):
            vec = plsc.load_gather(
                inb.at[buf],
                [lax.iota(jnp.int32, 16), jnp.full((16,), j, jnp.int32)])
            outb[buf, j // 8, pl.ds((j % 8) * 16, 16)] = vec
            return ()
        lax.fori_loop(0, 128, col, (), unroll=8)

    fire(tile0, 0)

    def body(i, _):
        cur = i % 2
        nxt = (i + 1) % 2

        @pl.when(i + 1 < ntiles)
        def _():
            fire(tile0 + i + 1, nxt)

        wait(cur)
        transpose_tile(cur)
        pltpu.sync_copy(outb.at[cur],
                        wp_hbm.at[pl.ds((tile0 + i) * 16, 16)])
        return ()

    lax.fori_loop(0, ntiles, body, ())

    @pl.when(wid == 0)
    def _():
        pltpu.sync_copy(tail_hbm, tailb)
        for k in range(64):
            outb[0, 0, pl.ds((k % 8) * 16, 16)] = tailb[k, :]
            if k % 8 == 7:
                pltpu.sync_copy(outb.at[0, 0],
                                wp_hbm.at[_FULL_TILES * 16 + k // 8])


@functools.partial(
    pl.kernel,
    mesh=plsc.VectorSubcoreMesh(core_axis_name="c", subcore_axis_name="s"),
    out_type=jax.ShapeDtypeStruct((_N_TABLES, _DIM, _BATCH), jnp.float32),
    scratch_types=[
        pltpu.VMEM((2, 128), jnp.int32),
        pltpu.VMEM((2, 128), jnp.int32),
        pltpu.VMEM((2, 128, 128), jnp.float32),
        pltpu.VMEM((2, 16, 128), jnp.float32),
        pltpu.SemaphoreType.DMA((2,)),
    ],
    compiler_params=pltpu.CompilerParams(use_tc_tiling_on_sc=True, needs_layout_passes=False),
)
def _gather(wp_hbm, idx_hbm, out_hbm, idxraw, idxrow, rowsv, outv, gsems):
    wid = lax.axis_index("s") * _NC + lax.axis_index("c")
    base = wid * _ROWS_PER_W

    def fire(blk, buf):
        pltpu.sync_copy(idx_hbm.at[blk], idxraw.at[buf])
        for kk in range(8):
            sl = pl.ds(kk * 16, 16)
            idxrow[buf, sl] = lax.shift_right_logical(idxraw[buf, sl], 3)
        pltpu.async_copy(wp_hbm.at[idxrow.at[buf]], rowsv.at[buf],
                         gsems.at[buf])

    def wait(buf):
        pltpu.make_async_copy(
            wp_hbm.at[idxrow.at[buf]], rowsv.at[buf], gsems.at[buf]).wait()

    def drain(blk, buf):
        t = blk // _CBLK
        c = blk % _CBLK
        wait(buf)

        rows2d = rowsv.at[buf]
        for j0 in range(0, 128, 16):
            vv = idxraw[buf, pl.ds(j0, 16)]
            colbase = (vv & 7) * 16
            rowix = lax.iota(jnp.int32, 16) + j0
            for d in range(_DIM):
                vec = plsc.load_gather(rows2d, [rowix, colbase + d])
                outv[buf, d, pl.ds(j0, 16)] = vec
        pltpu.sync_copy(
            outv.at[buf],
            out_hbm.at[t, :, pl.ds(pl.multiple_of(c * 128, 128), 128)])

    fire(base, 0)

    def body(i, _):
        cur = i % 2
        nxt = (i + 1) % 2

        @pl.when(i + 1 < _ROWS_PER_W)
        def _():
            fire(base + i + 1, nxt)

        drain(base + i, cur)
        return ()

    lax.fori_loop(0, _ROWS_PER_W, body, ())


def kernel(indices, offsets, weights):
    del offsets
    wt = weights.T
    tail = weights[_FULL_TILES * 128:, :]
    wp = _pack(wt, tail)
    idx2d = indices.astype(jnp.int32).reshape(_ROWS, _G)
    out = _gather(wp, idx2d)
    return out.transpose(0, 2, 1)


# speedup vs baseline: 1.4466x; 1.3139x over previous
import functools

import jax
import jax.numpy as jnp
from jax import lax
from jax.experimental import pallas as pl
from jax.experimental.pallas import tpu as pltpu
from jax.experimental.pallas import tpu_sc as plsc

_N_TABLES = 26
_BATCH = 16384
_DIM = 16
_B = _N_TABLES * _BATCH
_V = _N_TABLES * 100000

_NC = 2
_NS = 16
_NW = _NC * _NS

_FULL_TILES = _V // 128
_WP_ROWS = _V * _DIM // 128
_TILES_LO = _FULL_TILES // _NW
_TILES_EXTRA = _FULL_TILES - _TILES_LO * _NW

_G = 128
_ROWS = _B // _G
_ROWS_PER_W = _ROWS // _NW
_CBLK = _BATCH // _G


@functools.partial(
    pl.kernel,
    mesh=plsc.VectorSubcoreMesh(core_axis_name="c", subcore_axis_name="s"),
    out_type=jax.ShapeDtypeStruct((_WP_ROWS, 128), jnp.float32),
    scratch_types=[
        pltpu.VMEM((2, 16, 128), jnp.float32),
        pltpu.VMEM((2, 16, 128), jnp.float32),
        pltpu.VMEM((64, 16), jnp.float32),
        pltpu.SemaphoreType.DMA((2,)),
    ],
    compiler_params=pltpu.CompilerParams(use_tc_tiling_on_sc=True, needs_layout_passes=False),
)
def _pack(wt_hbm, tail_hbm, wp_hbm, inb, outb, tailb, isems):
    wid = lax.axis_index("s") * _NC + lax.axis_index("c")
    ntiles = jnp.where(wid < _TILES_EXTRA, _TILES_LO + 1, _TILES_LO)
    tile0 = wid * _TILES_LO + jnp.minimum(wid, _TILES_EXTRA)

    def fire(t, buf):
        pltpu.async_copy(
            wt_hbm.at[:, pl.ds(pl.multiple_of(t * 128, 128), 128)],
            inb.at[buf], isems.at[buf])

    def wait(buf):
        pltpu.make_async_copy(
            wt_hbm.at[:, pl.ds(0, 128)], inb.at[buf], isems.at[buf]).wait()

    iota16 = lax.iota(jnp.int32, 16)
    mod4 = iota16 & 3
    div4 = lax.shift_right_logical(iota16, 2)
    sc_base = div4 * 16 + mod4

    def transpose_tile(buf):
        src = inb.at[buf]
        dst = outb.at[buf]
        for k in range(32):
            j0 = 4 * k
            row = j0 // 8
            base = 16 * j0 - 128 * row
            for g in range(4):
                vec = plsc.load_gather(src, [mod4 + 4 * g, div4 + j0])
                plsc.store_scatter(
                    dst,
                    [jnp.full((16,), row, jnp.int32),
                     sc_base + (base + 4 * g)],
                    vec)

    fire(tile0, 0)

    def body(i, _):
        cur = i % 2
        nxt = (i + 1) % 2

        @pl.when(i + 1 < ntiles)
        def _():
            fire(tile0 + i + 1, nxt)

        wait(cur)
        transpose_tile(cur)
        pltpu.sync_copy(outb.at[cur],
                        wp_hbm.at[pl.ds((tile0 + i) * 16, 16)])
        return ()

    lax.fori_loop(0, ntiles, body, ())

    @pl.when(wid == 0)
    def _():
        pltpu.sync_copy(tail_hbm, tailb)
        for k in range(64):
            outb[0, 0, pl.ds((k % 8) * 16, 16)] = tailb[k, :]
            if k % 8 == 7:
                pltpu.sync_copy(outb.at[0, 0],
                                wp_hbm.at[_FULL_TILES * 16 + k // 8])


@functools.partial(
    pl.kernel,
    mesh=plsc.VectorSubcoreMesh(core_axis_name="c", subcore_axis_name="s"),
    out_type=jax.ShapeDtypeStruct((_N_TABLES, _DIM, _BATCH), jnp.float32),
    scratch_types=[
        pltpu.VMEM((2, 128), jnp.int32),
        pltpu.VMEM((2, 128), jnp.int32),
        pltpu.VMEM((2, 128, 128), jnp.float32),
        pltpu.VMEM((2, 16, 128), jnp.float32),
        pltpu.SemaphoreType.DMA((2,)),
    ],
    compiler_params=pltpu.CompilerParams(use_tc_tiling_on_sc=True, needs_layout_passes=False),
)
def _gather(wp_hbm, idx_hbm, out_hbm, idxraw, idxrow, rowsv, outv, gsems):
    wid = lax.axis_index("s") * _NC + lax.axis_index("c")
    base = wid * _ROWS_PER_W

    def fire(blk, buf):
        pltpu.sync_copy(idx_hbm.at[blk], idxraw.at[buf])
        for kk in range(8):
            sl = pl.ds(kk * 16, 16)
            idxrow[buf, sl] = lax.shift_right_logical(idxraw[buf, sl], 3)
        pltpu.async_copy(wp_hbm.at[idxrow.at[buf]], rowsv.at[buf],
                         gsems.at[buf])

    def wait(buf):
        pltpu.make_async_copy(
            wp_hbm.at[idxrow.at[buf]], rowsv.at[buf], gsems.at[buf]).wait()

    def drain(blk, buf):
        t = blk // _CBLK
        c = blk % _CBLK
        wait(buf)

        rows2d = rowsv.at[buf]
        for j0 in range(0, 128, 16):
            vv = idxraw[buf, pl.ds(j0, 16)]
            colbase = (vv & 7) * 16
            rowix = lax.iota(jnp.int32, 16) + j0
            for d in range(_DIM):
                vec = plsc.load_gather(rows2d, [rowix, colbase + d])
                outv[buf, d, pl.ds(j0, 16)] = vec
        pltpu.sync_copy(
            outv.at[buf],
            out_hbm.at[t, :, pl.ds(pl.multiple_of(c * 128, 128), 128)])

    fire(base, 0)

    def body(i, _):
        cur = i % 2
        nxt = (i + 1) % 2

        @pl.when(i + 1 < _ROWS_PER_W)
        def _():
            fire(base + i + 1, nxt)

        drain(base + i, cur)
        return ()

    lax.fori_loop(0, _ROWS_PER_W, body, ())


def kernel(indices, offsets, weights):
    del offsets
    wt = weights.T
    tail = weights[_FULL_TILES * 128:, :]
    wp = _pack(wt, tail)
    idx2d = indices.astype(jnp.int32).reshape(_ROWS, _G)
    out = _gather(wp, idx2d)
    return out.transpose(0, 2, 1)
